# Initial kernel scaffold; baseline (speedup 1.0000x reference)
import functools

import jax
import jax.numpy as jnp
from jax import lax
from jax.experimental import pallas as pl
from jax.experimental.pallas import tpu as pltpu
from jax.experimental.pallas import tpu_sc as plsc

NC = 2
NS = 16
NW = NC * NS

CHUNK = 256
GGRP = 128
KEB = 1024
NBLK = 10


def _f32(*shape):
    return jax.ShapeDtypeStruct(shape, jnp.float32)


def _edge_proj(ea_s, wm3t, bm):
    epad = ea_s.shape[0]
    de = ea_s.shape[1]

    def body(ea_ref, w_ref, b_ref, o_ref):
        o_ref[...] = (
            jnp.dot(ea_ref[...], w_ref[...], preferred_element_type=jnp.float32)
            + b_ref[...]
        )

    return pl.pallas_call(
        body,
        grid=(epad // KEB,),
        in_specs=[
            pl.BlockSpec((KEB, de), lambda i: (i, 0)),
            pl.BlockSpec((de, 128), lambda i: (0, 0)),
            pl.BlockSpec((1, 128), lambda i: (0, 0)),
        ],
        out_specs=pl.BlockSpec((KEB, 128), lambda i: (i, 0)),
        out_shape=_f32(epad, 128),
    )(ea_s, wm3t, bm)


def _tc_encode(state, hidden, wz1, wz2, be, wm1, wm2):
    n = state.shape[0]
    blk = n // NBLK

    def body(s_ref, h_ref, wz1_ref, wz2_ref, be_ref, wm1_ref, wm2_ref,
             z_ref, a_ref, b_ref):
        z = (
            jnp.dot(s_ref[...], wz1_ref[...], preferred_element_type=jnp.float32)
            + jnp.dot(h_ref[...], wz2_ref[...], preferred_element_type=jnp.float32)
            + be_ref[...]
        )
        z_ref[...] = z
        a_ref[...] = jnp.dot(z, wm1_ref[...], preferred_element_type=jnp.float32)
        b_ref[...] = jnp.dot(z, wm2_ref[...], preferred_element_type=jnp.float32)

    full = pl.BlockSpec((128, 128), lambda i: (0, 0))
    return pl.pallas_call(
        body,
        grid=(NBLK,),
        in_specs=[
            pl.BlockSpec((blk, 128), lambda i: (i, 0)),
            pl.BlockSpec((blk, 128), lambda i: (i, 0)),
            full, full,
            pl.BlockSpec((1, 128), lambda i: (0, 0)),
            full, full,
        ],
        out_specs=[
            pl.BlockSpec((blk, 128), lambda i: (i, 0)),
            pl.BlockSpec((blk, 128), lambda i: (i, 0)),
            pl.BlockSpec((blk, 128), lambda i: (i, 0)),
        ],
        out_shape=[_f32(n, 128), _f32(n, 128), _f32(n, 128)],
    )(state, hidden, wz1, wz2, be, wm1, wm2)


def _tc_update(s, z, b_rows, wu1, wu2, bu, wd1, wd2, bd):
    n = z.shape[0]
    blk = n // NBLK

    def body(s_ref, z_ref, brow_ref, wu1_ref, wu2_ref, bu_ref,
             wd1_ref, wd2_ref, bd_ref, nh_ref, ns_ref, hsum_ref):
        sblk = s_ref[...]
        u = jnp.where(sblk == -jnp.inf, 0.0, sblk + brow_ref[...])
        nh = (
            jnp.dot(z_ref[...], wu1_ref[...], preferred_element_type=jnp.float32)
            + jnp.dot(u, wu2_ref[...], preferred_element_type=jnp.float32)
            + bu_ref[...]
        )
        nh_ref[...] = nh
        ns_ref[...] = (
            jnp.dot(nh, wd1_ref[...], preferred_element_type=jnp.float32)
            + jnp.dot(z_ref[...], wd2_ref[...], preferred_element_type=jnp.float32)
            + bd_ref[...]
        )

        @pl.when(pl.program_id(0) == 0)
        def _():
            hsum_ref[...] = jnp.zeros_like(hsum_ref)

        hsum_ref[...] += jnp.sum(nh, axis=0, keepdims=True)

    full = pl.BlockSpec((128, 128), lambda i: (0, 0))
    rowspec = pl.BlockSpec((blk, 128), lambda i: (i, 0))
    return pl.pallas_call(
        body,
        grid=(NBLK,),
        in_specs=[
            rowspec, rowspec, rowspec,
            full, full, pl.BlockSpec((1, 128), lambda i: (0, 0)),
            full, full, pl.BlockSpec((1, 128), lambda i: (0, 0)),
        ],
        out_specs=[
            rowspec, rowspec,
            pl.BlockSpec((1, 128), lambda i: (0, 0)),
        ],
        out_shape=[_f32(n, 128), _f32(n, 128), _f32(1, 128)],
    )(s, z, b_rows, wu1, wu2, bu, wd1, wd2, bd)


def _tc_stop(nh, hsum, wt1, wt2, bt):
    n = nh.shape[0]
    blk = n // NBLK
    inv_n = 1.0 / n

    def body(nh_ref, hsum_ref, wt1_ref, wt2_ref, bt_ref, o_ref):
        m = hsum_ref[...] * inv_n
        v = (
            jnp.dot(nh_ref[...], wt1_ref[...], preferred_element_type=jnp.float32)
            + jnp.dot(m, wt2_ref[...], preferred_element_type=jnp.float32)
            + bt_ref[...]
        )
        o_ref[...] = 1.0 / (1.0 + jnp.exp(-v))

    return pl.pallas_call(
        body,
        grid=(NBLK,),
        in_specs=[
            pl.BlockSpec((blk, 128), lambda i: (i, 0)),
            pl.BlockSpec((1, 128), lambda i: (0, 0)),
            pl.BlockSpec((128, 2), lambda i: (0, 0)),
            pl.BlockSpec((128, 2), lambda i: (0, 0)),
            pl.BlockSpec((1, 2), lambda i: (0, 0)),
        ],
        out_specs=pl.BlockSpec((blk, 2), lambda i: (i, 0)),
        out_shape=_f32(n, 2),
    )(nh, hsum, wt1, wt2, bt)


def _sc_segmax(a_nodes, src_s, dst_s, e_s, offs, npt):
    n_pad = NW * npt
    mesh = plsc.VectorSubcoreMesh(core_axis_name="c", subcore_axis_name="s")

    @functools.partial(
        pl.kernel,
        out_type=_f32(n_pad, 128),
        mesh=mesh,
        scratch_types=[
            pltpu.VMEM((40,), jnp.int32),
            pltpu.VMEM((CHUNK,), jnp.int32),
            pltpu.VMEM((CHUNK,), jnp.int32),
            pltpu.VMEM((CHUNK, 128), jnp.float32),
            pltpu.VMEM((CHUNK, 128), jnp.float32),
            pltpu.VMEM((npt, 128), jnp.float32),
            pltpu.SemaphoreType.DMA,
        ],
    )
    def k(a_hbm, src_hbm, dst_hbm, e_hbm, off_hbm, out_hbm,
          off_v, src_v, dst_v, a_v, e_v, acc, sem):
        wid = lax.axis_index("s") * NC + lax.axis_index("c")
        lo = wid * npt
        pltpu.sync_copy(off_hbm, off_v)
        start = off_v[wid]
        end = off_v[wid + 1]
        astart = (start // 8) * 8
        nchunks = (end - astart + CHUNK - 1) // CHUNK

        neg_inf = jnp.full((16,), -jnp.inf, jnp.float32)

        def initrow(i, carry):
            for j in range(8):
                acc[i, pl.ds(j * 16, 16)] = neg_inf
            return carry

        lax.fori_loop(0, npt, initrow, 0)

        def chunk_body(c, carry):
            base = astart + c * CHUNK
            pltpu.sync_copy(src_hbm.at[pl.ds(base, CHUNK)], src_v)
            pltpu.sync_copy(dst_hbm.at[pl.ds(base, CHUNK)], dst_v)
            pltpu.sync_copy(e_hbm.at[pl.ds(base, CHUNK)], e_v)
            cps = []
            for g in range(CHUNK // GGRP):
                cps.append(
                    pltpu.async_copy(
                        a_hbm.at[src_v.at[pl.ds(g * GGRP, GGRP)]],
                        a_v.at[pl.ds(g * GGRP, GGRP)],
                        sem,
                    )
                )
            for cp in cps:
                cp.wait()
            lb = jnp.maximum(start - base, 0)
            ub = jnp.minimum(end - base, CHUNK)

            def edge_body(i, ecarry):
                d = dst_v[i] - lo
                for j in range(8):
                    sl = pl.ds(j * 16, 16)
                    p = a_v[i, sl] + e_v[i, sl]
                    acc[d, sl] = jnp.maximum(acc[d, sl], p)
                return ecarry

            lax.fori_loop(lb, ub, edge_body, 0)
            return carry

        lax.fori_loop(0, nchunks, chunk_body, 0)
        pltpu.sync_copy(acc, out_hbm.at[pl.ds(lo, npt)])

    return k(a_nodes, src_s, dst_s, e_s, offs)


def kernel(x, edge_index, edge_attr, W_enc, b_enc, W_M, b_M, W_U, b_U,
           W_dec, b_dec, W_term, b_term):
    n = x.shape[0]
    e = edge_attr.shape[0]
    d_hid = W_enc.shape[0]
    npt = -(-n // NW)

    src = edge_index[0].astype(jnp.int32)
    dst = edge_index[1].astype(jnp.int32)

    perm = jnp.argsort(dst)
    dst_s = dst[perm]
    src_s = src[perm]
    ea_s = edge_attr[perm]

    e_pad = ((e + 2 * CHUNK + KEB - 1) // KEB) * KEB
    big = jnp.int32(1 << 28)
    dst_s = jnp.concatenate([dst_s, jnp.full((e_pad - e,), big, jnp.int32)])
    src_s = jnp.concatenate([src_s, jnp.zeros((e_pad - e,), jnp.int32)])
    ea_s = jnp.concatenate(
        [ea_s, jnp.zeros((e_pad - e, ea_s.shape[1]), jnp.float32)]
    )
    tile_lo = (jnp.arange(33, dtype=jnp.int32) * npt).astype(jnp.int32)
    offs = jnp.searchsorted(dst_s, tile_lo).astype(jnp.int32)
    offs = jnp.concatenate([offs, jnp.full((7,), e, jnp.int32)])

    wz1 = W_enc[:, :d_hid].T
    wz2 = W_enc[:, d_hid:].T
    wm1 = W_M[:, :d_hid].T
    wm2 = W_M[:, d_hid : 2 * d_hid].T
    wm3 = W_M[:, 2 * d_hid :].T
    wu1 = W_U[:, :d_hid].T
    wu2 = W_U[:, d_hid:].T
    wd1 = W_dec[:, :d_hid].T
    wd2 = W_dec[:, d_hid:].T
    wt1 = W_term[:, :d_hid].T
    wt2 = W_term[:, d_hid:].T
    be = b_enc[None, :]
    bm = b_M[None, :]
    bu = b_U[None, :]
    bd = b_dec[None, :]
    bt = b_term[None, :]

    e_rows = _edge_proj(ea_s, wm3, bm)

    state = x
    hidden = jnp.zeros((n, d_hid), jnp.float32)
    preds = [x]
    stops = [jnp.zeros((n, 2), jnp.float32)]
    for _ in range(3):
        z, a_nodes, b_nodes = _tc_encode(state, hidden, wz1, wz2, be, wm1, wm2)
        s_full = _sc_segmax(a_nodes, src_s, dst_s, e_rows, offs, npt)
        s = s_full[:n]
        nh, ns, hsum = _tc_update(s, z, b_nodes, wu1, wu2, bu, wd1, wd2, bd)
        stop = _tc_stop(nh, hsum, wt1, wt2, bt)
        preds.append(ns)
        stops.append(stop)
        state, hidden = ns, nh

    return jnp.stack(preds, axis=1), jnp.stack(stops, axis=1)


# speedup vs baseline: 2.1145x; 2.1145x over previous
import functools

import jax
import jax.numpy as jnp
from jax import lax
from jax.experimental import pallas as pl
from jax.experimental.pallas import tpu as pltpu
from jax.experimental.pallas import tpu_sc as plsc

NC = 2
NS = 16
NW = NC * NS

CHUNK = 256
GGRP = 128
KEB = 1024
NBLK = 10


def _f32(*shape):
    return jax.ShapeDtypeStruct(shape, jnp.float32)


def _edge_proj(ea_s, wm3t, bm):
    epad = ea_s.shape[0]
    de = ea_s.shape[1]

    def body(ea_ref, w_ref, b_ref, o_ref):
        o_ref[...] = (
            jnp.dot(ea_ref[...], w_ref[...], preferred_element_type=jnp.float32)
            + b_ref[...]
        )

    return pl.pallas_call(
        body,
        grid=(epad // KEB,),
        in_specs=[
            pl.BlockSpec((KEB, de), lambda i: (i, 0)),
            pl.BlockSpec((de, 128), lambda i: (0, 0)),
            pl.BlockSpec((1, 128), lambda i: (0, 0)),
        ],
        out_specs=pl.BlockSpec((KEB, 128), lambda i: (i, 0)),
        out_shape=_f32(epad, 128),
    )(ea_s, wm3t, bm)


def _tc_encode(state, hidden, wz1, wz2, be, wm1, wm2):
    n = state.shape[0]
    blk = n // NBLK

    def body(s_ref, h_ref, wz1_ref, wz2_ref, be_ref, wm1_ref, wm2_ref,
             z_ref, a_ref, b_ref):
        z = (
            jnp.dot(s_ref[...], wz1_ref[...], preferred_element_type=jnp.float32)
            + jnp.dot(h_ref[...], wz2_ref[...], preferred_element_type=jnp.float32)
            + be_ref[...]
        )
        z_ref[...] = z
        a_ref[...] = jnp.dot(z, wm1_ref[...], preferred_element_type=jnp.float32)
        b_ref[...] = jnp.dot(z, wm2_ref[...], preferred_element_type=jnp.float32)

    full = pl.BlockSpec((128, 128), lambda i: (0, 0))
    return pl.pallas_call(
        body,
        grid=(NBLK,),
        in_specs=[
            pl.BlockSpec((blk, 128), lambda i: (i, 0)),
            pl.BlockSpec((blk, 128), lambda i: (i, 0)),
            full, full,
            pl.BlockSpec((1, 128), lambda i: (0, 0)),
            full, full,
        ],
        out_specs=[
            pl.BlockSpec((blk, 128), lambda i: (i, 0)),
            pl.BlockSpec((blk, 128), lambda i: (i, 0)),
            pl.BlockSpec((blk, 128), lambda i: (i, 0)),
        ],
        out_shape=[_f32(n, 128), _f32(n, 128), _f32(n, 128)],
    )(state, hidden, wz1, wz2, be, wm1, wm2)


def _tc_update(s, z, b_rows, wu1, wu2, bu, wd1, wd2, bd):
    n = z.shape[0]
    blk = n // NBLK

    def body(s_ref, z_ref, brow_ref, wu1_ref, wu2_ref, bu_ref,
             wd1_ref, wd2_ref, bd_ref, nh_ref, ns_ref, hsum_ref):
        sblk = s_ref[...]
        u = jnp.where(sblk == -jnp.inf, 0.0, sblk + brow_ref[...])
        nh = (
            jnp.dot(z_ref[...], wu1_ref[...], preferred_element_type=jnp.float32)
            + jnp.dot(u, wu2_ref[...], preferred_element_type=jnp.float32)
            + bu_ref[...]
        )
        nh_ref[...] = nh
        ns_ref[...] = (
            jnp.dot(nh, wd1_ref[...], preferred_element_type=jnp.float32)
            + jnp.dot(z_ref[...], wd2_ref[...], preferred_element_type=jnp.float32)
            + bd_ref[...]
        )

        @pl.when(pl.program_id(0) == 0)
        def _():
            hsum_ref[...] = jnp.zeros_like(hsum_ref)

        hsum_ref[...] += jnp.sum(nh, axis=0, keepdims=True)

    full = pl.BlockSpec((128, 128), lambda i: (0, 0))
    rowspec = pl.BlockSpec((blk, 128), lambda i: (i, 0))
    return pl.pallas_call(
        body,
        grid=(NBLK,),
        in_specs=[
            rowspec, rowspec, rowspec,
            full, full, pl.BlockSpec((1, 128), lambda i: (0, 0)),
            full, full, pl.BlockSpec((1, 128), lambda i: (0, 0)),
        ],
        out_specs=[
            rowspec, rowspec,
            pl.BlockSpec((1, 128), lambda i: (0, 0)),
        ],
        out_shape=[_f32(n, 128), _f32(n, 128), _f32(1, 128)],
    )(s, z, b_rows, wu1, wu2, bu, wd1, wd2, bd)


def _tc_stop(nh, hsum, wt1, wt2, bt):
    n = nh.shape[0]
    blk = n // NBLK
    inv_n = 1.0 / n

    def body(nh_ref, hsum_ref, wt1_ref, wt2_ref, bt_ref, o_ref):
        m = hsum_ref[...] * inv_n
        v = (
            jnp.dot(nh_ref[...], wt1_ref[...], preferred_element_type=jnp.float32)
            + jnp.dot(m, wt2_ref[...], preferred_element_type=jnp.float32)
            + bt_ref[...]
        )
        o_ref[...] = 1.0 / (1.0 + jnp.exp(-v))

    return pl.pallas_call(
        body,
        grid=(NBLK,),
        in_specs=[
            pl.BlockSpec((blk, 128), lambda i: (i, 0)),
            pl.BlockSpec((1, 128), lambda i: (0, 0)),
            pl.BlockSpec((128, 2), lambda i: (0, 0)),
            pl.BlockSpec((128, 2), lambda i: (0, 0)),
            pl.BlockSpec((1, 2), lambda i: (0, 0)),
        ],
        out_specs=pl.BlockSpec((blk, 2), lambda i: (i, 0)),
        out_shape=_f32(n, 2),
    )(nh, hsum, wt1, wt2, bt)


def _sc_segmax(a_nodes, src_s, dst_s, e_s, offs, npt):
    n_pad = NW * npt
    mesh = plsc.VectorSubcoreMesh(core_axis_name="c", subcore_axis_name="s")

    @functools.partial(
        pl.kernel,
        out_type=_f32(n_pad, 128),
        mesh=mesh,
        scratch_types=[
            pltpu.VMEM((48,), jnp.int32),
            pltpu.VMEM((CHUNK,), jnp.int32),
            pltpu.VMEM((CHUNK + 16,), jnp.int32),
            pltpu.VMEM((CHUNK, 128), jnp.float32),
            pltpu.VMEM((CHUNK, 128), jnp.float32),
            pltpu.VMEM((npt, 128), jnp.float32),
            pltpu.SemaphoreType.DMA,
        ],
    )
    def k(a_hbm, src_hbm, dst_hbm, e_hbm, off_hbm, out_hbm,
          off_v, src_v, dst_v, a_v, e_v, acc, sem):
        wid = lax.axis_index("s") * NC + lax.axis_index("c")
        lo = wid * npt
        pltpu.sync_copy(off_hbm, off_v)
        ov = off_v[pl.ds(wid, 16)]
        start = ov[0]
        end = ov[1]
        astart = (start // 8) * 8
        nchunks = (end - astart + CHUNK - 1) // CHUNK

        neg_inf = jnp.full((16,), -jnp.inf, jnp.float32)

        def initrow(i, carry):
            for j in range(8):
                acc[i, pl.ds(j * 16, 16)] = neg_inf
            return carry

        lax.fori_loop(0, npt, initrow, 0)

        def chunk_body(c, carry):
            base = astart + c * CHUNK
            pltpu.sync_copy(src_hbm.at[pl.ds(base, CHUNK)], src_v)
            pltpu.sync_copy(dst_hbm.at[pl.ds(base, CHUNK)], dst_v.at[pl.ds(0, CHUNK)])
            pltpu.sync_copy(e_hbm.at[pl.ds(base, CHUNK)], e_v)
            cps = []
            for g in range(CHUNK // GGRP):
                cps.append(
                    pltpu.async_copy(
                        a_hbm.at[src_v.at[pl.ds(g * GGRP, GGRP)]],
                        a_v.at[pl.ds(g * GGRP, GGRP)],
                        sem,
                    )
                )
            for cp in cps:
                cp.wait()
            lb = jnp.maximum(start - base, 0)
            ub = jnp.minimum(end - base, CHUNK)

            def edge_body(i, ecarry):
                d = dst_v[pl.ds(i, 16)][0] - lo
                for j in range(8):
                    sl = pl.ds(j * 16, 16)
                    p = a_v[i, sl] + e_v[i, sl]
                    acc[d, sl] = jnp.maximum(acc[d, sl], p)
                return ecarry

            lax.fori_loop(lb, ub, edge_body, 0)
            return carry

        lax.fori_loop(0, nchunks, chunk_body, 0)
        pltpu.sync_copy(acc, out_hbm.at[pl.ds(lo, npt)])

    return k(a_nodes, src_s, dst_s, e_s, offs)


def kernel(x, edge_index, edge_attr, W_enc, b_enc, W_M, b_M, W_U, b_U,
           W_dec, b_dec, W_term, b_term):
    n = x.shape[0]
    e = edge_attr.shape[0]
    d_hid = W_enc.shape[0]
    npt = ((-(-n // NW) + 7) // 8) * 8

    src = edge_index[0].astype(jnp.int32)
    dst = edge_index[1].astype(jnp.int32)

    perm = jnp.argsort(dst)
    dst_s = dst[perm]
    src_s = src[perm]
    ea_s = edge_attr[perm]

    e_pad = ((e + 2 * CHUNK + KEB - 1) // KEB) * KEB
    big = jnp.int32(1 << 28)
    dst_s = jnp.concatenate([dst_s, jnp.full((e_pad - e,), big, jnp.int32)])
    src_s = jnp.concatenate([src_s, jnp.zeros((e_pad - e,), jnp.int32)])
    ea_s = jnp.concatenate(
        [ea_s, jnp.zeros((e_pad - e, ea_s.shape[1]), jnp.float32)]
    )
    tile_lo = (jnp.arange(33, dtype=jnp.int32) * npt).astype(jnp.int32)
    offs = jnp.searchsorted(dst_s, tile_lo).astype(jnp.int32)
    offs = jnp.concatenate([offs, jnp.full((15,), e, jnp.int32)])

    wz1 = W_enc[:, :d_hid].T
    wz2 = W_enc[:, d_hid:].T
    wm1 = W_M[:, :d_hid].T
    wm2 = W_M[:, d_hid : 2 * d_hid].T
    wm3 = W_M[:, 2 * d_hid :].T
    wu1 = W_U[:, :d_hid].T
    wu2 = W_U[:, d_hid:].T
    wd1 = W_dec[:, :d_hid].T
    wd2 = W_dec[:, d_hid:].T
    wt1 = W_term[:, :d_hid].T
    wt2 = W_term[:, d_hid:].T
    be = b_enc[None, :]
    bm = b_M[None, :]
    bu = b_U[None, :]
    bd = b_dec[None, :]
    bt = b_term[None, :]

    e_rows = _edge_proj(ea_s, wm3, bm)

    state = x
    hidden = jnp.zeros((n, d_hid), jnp.float32)
    preds = [x]
    stops = [jnp.zeros((n, 2), jnp.float32)]
    for _ in range(3):
        z, a_nodes, b_nodes = _tc_encode(state, hidden, wz1, wz2, be, wm1, wm2)
        s_full = _sc_segmax(a_nodes, src_s, dst_s, e_rows, offs, npt)
        s = s_full[:n]
        nh, ns, hsum = _tc_update(s, z, b_nodes, wu1, wu2, bu, wd1, wd2, bd)
        stop = _tc_stop(nh, hsum, wt1, wt2, bt)
        preds.append(ns)
        stops.append(stop)
        state, hidden = ns, nh

    return jnp.stack(preds, axis=1), jnp.stack(stops, axis=1)


# speedup vs baseline: 2.9766x; 1.4077x over previous
import functools

import jax
import jax.numpy as jnp
from jax import lax
from jax.experimental import pallas as pl
from jax.experimental.pallas import tpu as pltpu
from jax.experimental.pallas import tpu_sc as plsc

NC = 2
NS = 16
NW = NC * NS

CHUNK = 256
GGRP = 128
KEB = 1024
NBLK = 10


def _f32(*shape):
    return jax.ShapeDtypeStruct(shape, jnp.float32)


def _edge_proj(ea_s, wm3t, bm):
    epad = ea_s.shape[0]
    de = ea_s.shape[1]

    def body(ea_ref, w_ref, b_ref, o_ref):
        o_ref[...] = (
            jnp.dot(ea_ref[...], w_ref[...], preferred_element_type=jnp.float32)
            + b_ref[...]
        )

    return pl.pallas_call(
        body,
        grid=(epad // KEB,),
        in_specs=[
            pl.BlockSpec((KEB, de), lambda i: (i, 0)),
            pl.BlockSpec((de, 128), lambda i: (0, 0)),
            pl.BlockSpec((1, 128), lambda i: (0, 0)),
        ],
        out_specs=pl.BlockSpec((KEB, 128), lambda i: (i, 0)),
        out_shape=_f32(epad, 128),
    )(ea_s, wm3t, bm)


def _tc_encode(state, hidden, wz1, wz2, be, wm1, wm2):
    n = state.shape[0]
    blk = n // NBLK

    def body(s_ref, h_ref, wz1_ref, wz2_ref, be_ref, wm1_ref, wm2_ref,
             z_ref, a_ref, b_ref):
        z = (
            jnp.dot(s_ref[...], wz1_ref[...], preferred_element_type=jnp.float32)
            + jnp.dot(h_ref[...], wz2_ref[...], preferred_element_type=jnp.float32)
            + be_ref[...]
        )
        z_ref[...] = z
        a_ref[...] = jnp.dot(z, wm1_ref[...], preferred_element_type=jnp.float32)
        b_ref[...] = jnp.dot(z, wm2_ref[...], preferred_element_type=jnp.float32)

    full = pl.BlockSpec((128, 128), lambda i: (0, 0))
    return pl.pallas_call(
        body,
        grid=(NBLK,),
        in_specs=[
            pl.BlockSpec((blk, 128), lambda i: (i, 0)),
            pl.BlockSpec((blk, 128), lambda i: (i, 0)),
            full, full,
            pl.BlockSpec((1, 128), lambda i: (0, 0)),
            full, full,
        ],
        out_specs=[
            pl.BlockSpec((blk, 128), lambda i: (i, 0)),
            pl.BlockSpec((blk, 128), lambda i: (i, 0)),
            pl.BlockSpec((blk, 128), lambda i: (i, 0)),
        ],
        out_shape=[_f32(n, 128), _f32(n, 128), _f32(n, 128)],
    )(state, hidden, wz1, wz2, be, wm1, wm2)


def _tc_update(s, z, b_rows, wu1, wu2, bu, wd1, wd2, bd):
    n = z.shape[0]
    blk = n // NBLK

    def body(s_ref, z_ref, brow_ref, wu1_ref, wu2_ref, bu_ref,
             wd1_ref, wd2_ref, bd_ref, nh_ref, ns_ref, hsum_ref):
        sblk = s_ref[...]
        u = jnp.where(sblk == -jnp.inf, 0.0, sblk + brow_ref[...])
        nh = (
            jnp.dot(z_ref[...], wu1_ref[...], preferred_element_type=jnp.float32)
            + jnp.dot(u, wu2_ref[...], preferred_element_type=jnp.float32)
            + bu_ref[...]
        )
        nh_ref[...] = nh
        ns_ref[...] = (
            jnp.dot(nh, wd1_ref[...], preferred_element_type=jnp.float32)
            + jnp.dot(z_ref[...], wd2_ref[...], preferred_element_type=jnp.float32)
            + bd_ref[...]
        )

        @pl.when(pl.program_id(0) == 0)
        def _():
            hsum_ref[...] = jnp.zeros_like(hsum_ref)

        hsum_ref[...] += jnp.sum(nh, axis=0, keepdims=True)

    full = pl.BlockSpec((128, 128), lambda i: (0, 0))
    rowspec = pl.BlockSpec((blk, 128), lambda i: (i, 0))
    return pl.pallas_call(
        body,
        grid=(NBLK,),
        in_specs=[
            rowspec, rowspec, rowspec,
            full, full, pl.BlockSpec((1, 128), lambda i: (0, 0)),
            full, full, pl.BlockSpec((1, 128), lambda i: (0, 0)),
        ],
        out_specs=[
            rowspec, rowspec,
            pl.BlockSpec((1, 128), lambda i: (0, 0)),
        ],
        out_shape=[_f32(n, 128), _f32(n, 128), _f32(1, 128)],
    )(s, z, b_rows, wu1, wu2, bu, wd1, wd2, bd)


def _tc_stop(nh, hsum, wt1, wt2, bt):
    n = nh.shape[0]
    blk = n // NBLK
    inv_n = 1.0 / n

    def body(nh_ref, hsum_ref, wt1_ref, wt2_ref, bt_ref, o_ref):
        m = hsum_ref[...] * inv_n
        v = (
            jnp.dot(nh_ref[...], wt1_ref[...], preferred_element_type=jnp.float32)
            + jnp.dot(m, wt2_ref[...], preferred_element_type=jnp.float32)
            + bt_ref[...]
        )
        o_ref[...] = 1.0 / (1.0 + jnp.exp(-v))

    return pl.pallas_call(
        body,
        grid=(NBLK,),
        in_specs=[
            pl.BlockSpec((blk, 128), lambda i: (i, 0)),
            pl.BlockSpec((1, 128), lambda i: (0, 0)),
            pl.BlockSpec((128, 2), lambda i: (0, 0)),
            pl.BlockSpec((128, 2), lambda i: (0, 0)),
            pl.BlockSpec((1, 2), lambda i: (0, 0)),
        ],
        out_specs=pl.BlockSpec((blk, 2), lambda i: (i, 0)),
        out_shape=_f32(n, 2),
    )(nh, hsum, wt1, wt2, bt)


def _sc_segmax(a_nodes, src_s, dst_s, e_s, offs, npt):
    n_pad = NW * npt
    mesh = plsc.VectorSubcoreMesh(core_axis_name="c", subcore_axis_name="s")

    @functools.partial(
        pl.kernel,
        out_type=_f32(n_pad, 128),
        mesh=mesh,
        scratch_types=[
            pltpu.VMEM((48,), jnp.int32),
            pltpu.VMEM((CHUNK,), jnp.int32),
            pltpu.VMEM((CHUNK + 16,), jnp.int32),
            pltpu.VMEM((CHUNK, 128), jnp.float32),
            pltpu.VMEM((CHUNK, 128), jnp.float32),
            pltpu.VMEM((npt + 8, 128), jnp.float32),
            pltpu.SemaphoreType.DMA,
        ],
    )
    def k(a_hbm, src_hbm, dst_hbm, e_hbm, off_hbm, out_hbm,
          off_v, src_v, dst_v, a_v, e_v, acc, sem):
        wid = lax.axis_index("s") * NC + lax.axis_index("c")
        lo = wid * npt
        pltpu.sync_copy(off_hbm, off_v)
        ov = off_v[pl.ds(wid, 16)]
        start = ov[0]
        end = ov[1]
        astart = (start // 8) * 8
        nchunks = (end - astart + CHUNK - 1) // CHUNK

        neg_inf = jnp.full((16,), -jnp.inf, jnp.float32)

        def initrow(i, carry):
            for j in range(8):
                acc[i, pl.ds(j * 16, 16)] = neg_inf
            return carry

        lax.fori_loop(0, npt + 8, initrow, 0)

        init_carry = (jnp.int32(npt), *([neg_inf] * 8))

        def chunk_body(c, carry):
            base = astart + c * CHUNK
            pltpu.sync_copy(src_hbm.at[pl.ds(base, CHUNK)], src_v)
            pltpu.sync_copy(dst_hbm.at[pl.ds(base, CHUNK)], dst_v.at[pl.ds(0, CHUNK)])
            pltpu.sync_copy(e_hbm.at[pl.ds(base, CHUNK)], e_v)
            cps = []
            for g in range(CHUNK // GGRP):
                cps.append(
                    pltpu.async_copy(
                        a_hbm.at[src_v.at[pl.ds(g * GGRP, GGRP)]],
                        a_v.at[pl.ds(g * GGRP, GGRP)],
                        sem,
                    )
                )
            for cp in cps:
                cp.wait()
            lb = jnp.maximum(start - base, 0)
            ub = jnp.minimum(end - base, CHUNK)

            def edge_body(i, ecarry):
                d_prev = ecarry[0]
                m = ecarry[1:]
                d = dst_v[pl.ds(i, 16)][0] - lo
                same = d == d_prev

                @pl.when(jnp.logical_not(same))
                def _():
                    for j in range(8):
                        sl = pl.ds(j * 16, 16)
                        acc[d_prev, sl] = jnp.maximum(acc[d_prev, sl], m[j])

                new_m = []
                for j in range(8):
                    sl = pl.ds(j * 16, 16)
                    p = a_v[i, sl] + e_v[i, sl]
                    prev = jnp.where(same, m[j], neg_inf)
                    new_m.append(jnp.maximum(prev, p))
                return (d, *new_m)

            return lax.fori_loop(lb, ub, edge_body, carry)

        fin = lax.fori_loop(0, nchunks, chunk_body, init_carry)
        d_last = fin[0]
        for j in range(8):
            sl = pl.ds(j * 16, 16)
            acc[d_last, sl] = jnp.maximum(acc[d_last, sl], fin[1 + j])
        pltpu.sync_copy(acc.at[pl.ds(0, npt)], out_hbm.at[pl.ds(lo, npt)])

    return k(a_nodes, src_s, dst_s, e_s, offs)


def kernel(x, edge_index, edge_attr, W_enc, b_enc, W_M, b_M, W_U, b_U,
           W_dec, b_dec, W_term, b_term):
    n = x.shape[0]
    e = edge_attr.shape[0]
    d_hid = W_enc.shape[0]
    npt = ((-(-n // NW) + 7) // 8) * 8

    src = edge_index[0].astype(jnp.int32)
    dst = edge_index[1].astype(jnp.int32)

    perm = jnp.argsort(dst)
    dst_s = dst[perm]
    src_s = src[perm]
    ea_s = edge_attr[perm]

    e_pad = ((e + 2 * CHUNK + KEB - 1) // KEB) * KEB
    big = jnp.int32(1 << 28)
    dst_s = jnp.concatenate([dst_s, jnp.full((e_pad - e,), big, jnp.int32)])
    src_s = jnp.concatenate([src_s, jnp.zeros((e_pad - e,), jnp.int32)])
    ea_s = jnp.concatenate(
        [ea_s, jnp.zeros((e_pad - e, ea_s.shape[1]), jnp.float32)]
    )
    tile_lo = (jnp.arange(33, dtype=jnp.int32) * npt).astype(jnp.int32)
    offs = jnp.searchsorted(dst_s, tile_lo).astype(jnp.int32)
    offs = jnp.concatenate([offs, jnp.full((15,), e, jnp.int32)])

    wz1 = W_enc[:, :d_hid].T
    wz2 = W_enc[:, d_hid:].T
    wm1 = W_M[:, :d_hid].T
    wm2 = W_M[:, d_hid : 2 * d_hid].T
    wm3 = W_M[:, 2 * d_hid :].T
    wu1 = W_U[:, :d_hid].T
    wu2 = W_U[:, d_hid:].T
    wd1 = W_dec[:, :d_hid].T
    wd2 = W_dec[:, d_hid:].T
    wt1 = W_term[:, :d_hid].T
    wt2 = W_term[:, d_hid:].T
    be = b_enc[None, :]
    bm = b_M[None, :]
    bu = b_U[None, :]
    bd = b_dec[None, :]
    bt = b_term[None, :]

    e_rows = _edge_proj(ea_s, wm3, bm)

    state = x
    hidden = jnp.zeros((n, d_hid), jnp.float32)
    preds = [x]
    stops = [jnp.zeros((n, 2), jnp.float32)]
    for _ in range(3):
        z, a_nodes, b_nodes = _tc_encode(state, hidden, wz1, wz2, be, wm1, wm2)
        s_full = _sc_segmax(a_nodes, src_s, dst_s, e_rows, offs, npt)
        s = s_full[:n]
        nh, ns, hsum = _tc_update(s, z, b_nodes, wu1, wu2, bu, wd1, wd2, bd)
        stop = _tc_stop(nh, hsum, wt1, wt2, bt)
        preds.append(ns)
        stops.append(stop)
        state, hidden = ns, nh

    return jnp.stack(preds, axis=1), jnp.stack(stops, axis=1)


# speedup vs baseline: 3.3339x; 1.1200x over previous
import functools

import jax
import jax.numpy as jnp
from jax import lax
from jax.experimental import pallas as pl
from jax.experimental.pallas import tpu as pltpu
from jax.experimental.pallas import tpu_sc as plsc

NC = 2
NS = 16
NW = NC * NS

CHUNK = 128
KEB = 1024
NBLK = 10


def _f32(*shape):
    return jax.ShapeDtypeStruct(shape, jnp.float32)


def _edge_proj(ea_s, wm3t, bm):
    epad = ea_s.shape[0]
    de = ea_s.shape[1]

    def body(ea_ref, w_ref, b_ref, o_ref):
        o_ref[...] = (
            jnp.dot(ea_ref[...], w_ref[...], preferred_element_type=jnp.float32)
            + b_ref[...]
        )

    return pl.pallas_call(
        body,
        grid=(epad // KEB,),
        in_specs=[
            pl.BlockSpec((KEB, de), lambda i: (i, 0)),
            pl.BlockSpec((de, 128), lambda i: (0, 0)),
            pl.BlockSpec((1, 128), lambda i: (0, 0)),
        ],
        out_specs=pl.BlockSpec((KEB, 128), lambda i: (i, 0)),
        out_shape=_f32(epad, 128),
    )(ea_s, wm3t, bm)


def _tc_encode(state, hidden, wz1, wz2, be, wm1, wm2):
    n = state.shape[0]
    blk = n // NBLK

    def body(s_ref, h_ref, wz1_ref, wz2_ref, be_ref, wm1_ref, wm2_ref,
             z_ref, a_ref, b_ref):
        z = (
            jnp.dot(s_ref[...], wz1_ref[...], preferred_element_type=jnp.float32)
            + jnp.dot(h_ref[...], wz2_ref[...], preferred_element_type=jnp.float32)
            + be_ref[...]
        )
        z_ref[...] = z
        a_ref[...] = jnp.dot(z, wm1_ref[...], preferred_element_type=jnp.float32)
        b_ref[...] = jnp.dot(z, wm2_ref[...], preferred_element_type=jnp.float32)

    full = pl.BlockSpec((128, 128), lambda i: (0, 0))
    return pl.pallas_call(
        body,
        grid=(NBLK,),
        in_specs=[
            pl.BlockSpec((blk, 128), lambda i: (i, 0)),
            pl.BlockSpec((blk, 128), lambda i: (i, 0)),
            full, full,
            pl.BlockSpec((1, 128), lambda i: (0, 0)),
            full, full,
        ],
        out_specs=[
            pl.BlockSpec((blk, 128), lambda i: (i, 0)),
            pl.BlockSpec((blk, 128), lambda i: (i, 0)),
            pl.BlockSpec((blk, 128), lambda i: (i, 0)),
        ],
        out_shape=[_f32(n, 128), _f32(n, 128), _f32(n, 128)],
    )(state, hidden, wz1, wz2, be, wm1, wm2)


def _tc_update(s, z, b_rows, wu1, wu2, bu, wd1, wd2, bd):
    n = z.shape[0]
    blk = n // NBLK

    def body(s_ref, z_ref, brow_ref, wu1_ref, wu2_ref, bu_ref,
             wd1_ref, wd2_ref, bd_ref, nh_ref, ns_ref, hsum_ref):
        sblk = s_ref[...]
        u = jnp.where(sblk == -jnp.inf, 0.0, sblk + brow_ref[...])
        nh = (
            jnp.dot(z_ref[...], wu1_ref[...], preferred_element_type=jnp.float32)
            + jnp.dot(u, wu2_ref[...], preferred_element_type=jnp.float32)
            + bu_ref[...]
        )
        nh_ref[...] = nh
        ns_ref[...] = (
            jnp.dot(nh, wd1_ref[...], preferred_element_type=jnp.float32)
            + jnp.dot(z_ref[...], wd2_ref[...], preferred_element_type=jnp.float32)
            + bd_ref[...]
        )

        @pl.when(pl.program_id(0) == 0)
        def _():
            hsum_ref[...] = jnp.zeros_like(hsum_ref)

        hsum_ref[...] += jnp.sum(nh, axis=0, keepdims=True)

    full = pl.BlockSpec((128, 128), lambda i: (0, 0))
    rowspec = pl.BlockSpec((blk, 128), lambda i: (i, 0))
    return pl.pallas_call(
        body,
        grid=(NBLK,),
        in_specs=[
            rowspec, rowspec, rowspec,
            full, full, pl.BlockSpec((1, 128), lambda i: (0, 0)),
            full, full, pl.BlockSpec((1, 128), lambda i: (0, 0)),
        ],
        out_specs=[
            rowspec, rowspec,
            pl.BlockSpec((1, 128), lambda i: (0, 0)),
        ],
        out_shape=[_f32(n, 128), _f32(n, 128), _f32(1, 128)],
    )(s, z, b_rows, wu1, wu2, bu, wd1, wd2, bd)


def _tc_stop(nh, hsum, wt1, wt2, bt):
    n = nh.shape[0]
    blk = n // NBLK
    inv_n = 1.0 / n

    def body(nh_ref, hsum_ref, wt1_ref, wt2_ref, bt_ref, o_ref):
        m = hsum_ref[...] * inv_n
        v = (
            jnp.dot(nh_ref[...], wt1_ref[...], preferred_element_type=jnp.float32)
            + jnp.dot(m, wt2_ref[...], preferred_element_type=jnp.float32)
            + bt_ref[...]
        )
        o_ref[...] = 1.0 / (1.0 + jnp.exp(-v))

    return pl.pallas_call(
        body,
        grid=(NBLK,),
        in_specs=[
            pl.BlockSpec((blk, 128), lambda i: (i, 0)),
            pl.BlockSpec((1, 128), lambda i: (0, 0)),
            pl.BlockSpec((128, 2), lambda i: (0, 0)),
            pl.BlockSpec((128, 2), lambda i: (0, 0)),
            pl.BlockSpec((1, 2), lambda i: (0, 0)),
        ],
        out_specs=pl.BlockSpec((blk, 2), lambda i: (i, 0)),
        out_shape=_f32(n, 2),
    )(nh, hsum, wt1, wt2, bt)


def _sc_segmax(a_nodes, src_s, dst_s, e_s, offs, npt):
    n_pad = NW * npt
    mesh = plsc.VectorSubcoreMesh(core_axis_name="c", subcore_axis_name="s")

    @functools.partial(
        pl.kernel,
        out_type=_f32(n_pad, 128),
        mesh=mesh,
        scratch_types=[
            pltpu.VMEM((48,), jnp.int32),
            [pltpu.VMEM((CHUNK,), jnp.int32)] * 2,
            [pltpu.VMEM((CHUNK + 16,), jnp.int32)] * 2,
            [pltpu.VMEM((CHUNK, 128), jnp.float32)] * 2,
            [pltpu.VMEM((CHUNK, 128), jnp.float32)] * 2,
            pltpu.VMEM((npt + 8, 128), jnp.float32),
            [pltpu.SemaphoreType.DMA] * 2,
            [pltpu.SemaphoreType.DMA] * 2,
        ],
    )
    def k(a_hbm, src_hbm, dst_hbm, e_hbm, off_hbm, out_hbm,
          off_v, src_v, dst_v, a_v, e_v, acc, lin_sem, gat_sem):
        wid = lax.axis_index("s") * NC + lax.axis_index("c")
        lo = wid * npt
        pltpu.sync_copy(off_hbm, off_v)
        ov = off_v[pl.ds(wid, 16)]
        start = ov[0]
        end = ov[1]
        astart = (start // 8) * 8
        nchunks = (end - astart + CHUNK - 1) // CHUNK

        neg_inf = jnp.full((16,), -jnp.inf, jnp.float32)

        def initrow(i, carry):
            for j in range(8):
                acc[i, pl.ds(j * 16, 16)] = neg_inf
            return carry

        lax.fori_loop(0, npt + 8, initrow, 0)

        init_carry = (jnp.int32(npt), *([neg_inf] * 8))

        def issue_lin(c, b):
            @pl.when(c < nchunks)
            def _():
                base = astart + c * CHUNK
                pltpu.async_copy(src_hbm.at[pl.ds(base, CHUNK)], src_v[b], lin_sem[b])
                pltpu.async_copy(
                    dst_hbm.at[pl.ds(base, CHUNK)], dst_v[b].at[pl.ds(0, CHUNK)],
                    lin_sem[b],
                )
                pltpu.async_copy(e_hbm.at[pl.ds(base, CHUNK)], e_v[b], lin_sem[b])

        def wait_lin(c, b):
            @pl.when(c < nchunks)
            def _():
                pltpu.make_async_copy(
                    src_hbm.at[pl.ds(0, CHUNK)], src_v[b], lin_sem[b]
                ).wait()
                pltpu.make_async_copy(
                    dst_hbm.at[pl.ds(0, CHUNK)], dst_v[b].at[pl.ds(0, CHUNK)],
                    lin_sem[b],
                ).wait()
                pltpu.make_async_copy(
                    e_hbm.at[pl.ds(0, CHUNK)], e_v[b], lin_sem[b]
                ).wait()

        def issue_gat(c, b):
            @pl.when(c < nchunks)
            def _():
                pltpu.async_copy(a_hbm.at[src_v[b]], a_v[b], gat_sem[b])

        def wait_gat(c, b):
            @pl.when(c < nchunks)
            def _():
                pltpu.make_async_copy(
                    e_hbm.at[pl.ds(0, CHUNK)], a_v[b], gat_sem[b]
                ).wait()

        def process(c, b, carry):
            base = astart + c * CHUNK
            lb = jnp.maximum(start - base, 0)
            ub = jnp.minimum(end - base, CHUNK)
            dv, av, ev = dst_v[b], a_v[b], e_v[b]

            def edge_body(i, ecarry):
                d_prev = ecarry[0]
                m = ecarry[1:]
                d = dv[pl.ds(i, 16)][0] - lo
                same = d == d_prev

                @pl.when(jnp.logical_not(same))
                def _():
                    for j in range(8):
                        sl = pl.ds(j * 16, 16)
                        acc[d_prev, sl] = jnp.maximum(acc[d_prev, sl], m[j])

                new_m = []
                for j in range(8):
                    sl = pl.ds(j * 16, 16)
                    p = av[i, sl] + ev[i, sl]
                    prev = jnp.where(same, m[j], neg_inf)
                    new_m.append(jnp.maximum(prev, p))
                return (d, *new_m)

            return lax.fori_loop(lb, ub, edge_body, carry)

        issue_lin(0, 0)
        wait_lin(0, 0)
        issue_gat(0, 0)
        issue_lin(1, 1)
        nq = (nchunks + 1) // 2

        def pair_body(q, carry):
            c0 = 2 * q
            wait_gat(c0, 0)
            wait_lin(c0 + 1, 1)
            issue_gat(c0 + 1, 1)
            carry = process(c0, 0, carry)
            issue_lin(c0 + 2, 0)
            wait_gat(c0 + 1, 1)
            wait_lin(c0 + 2, 0)
            issue_gat(c0 + 2, 0)
            carry = process(c0 + 1, 1, carry)
            issue_lin(c0 + 3, 1)
            return carry

        fin = lax.fori_loop(0, nq, pair_body, init_carry)
        d_last = fin[0]
        for j in range(8):
            sl = pl.ds(j * 16, 16)
            acc[d_last, sl] = jnp.maximum(acc[d_last, sl], fin[1 + j])
        pltpu.sync_copy(acc.at[pl.ds(0, npt)], out_hbm.at[pl.ds(lo, npt)])

    return k(a_nodes, src_s, dst_s, e_s, offs)


def kernel(x, edge_index, edge_attr, W_enc, b_enc, W_M, b_M, W_U, b_U,
           W_dec, b_dec, W_term, b_term):
    n = x.shape[0]
    e = edge_attr.shape[0]
    d_hid = W_enc.shape[0]
    npt = ((-(-n // NW) + 7) // 8) * 8

    src = edge_index[0].astype(jnp.int32)
    dst = edge_index[1].astype(jnp.int32)

    perm = jnp.argsort(dst)
    dst_s = dst[perm]
    src_s = src[perm]
    ea_s = edge_attr[perm]

    e_pad = ((e + 2 * CHUNK + KEB - 1) // KEB) * KEB
    big = jnp.int32(1 << 28)
    dst_s = jnp.concatenate([dst_s, jnp.full((e_pad - e,), big, jnp.int32)])
    src_s = jnp.concatenate([src_s, jnp.zeros((e_pad - e,), jnp.int32)])
    ea_s = jnp.concatenate(
        [ea_s, jnp.zeros((e_pad - e, ea_s.shape[1]), jnp.float32)]
    )
    tile_lo = (jnp.arange(33, dtype=jnp.int32) * npt).astype(jnp.int32)
    offs = jnp.searchsorted(dst_s, tile_lo).astype(jnp.int32)
    offs = jnp.concatenate([offs, jnp.full((15,), e, jnp.int32)])

    wz1 = W_enc[:, :d_hid].T
    wz2 = W_enc[:, d_hid:].T
    wm1 = W_M[:, :d_hid].T
    wm2 = W_M[:, d_hid : 2 * d_hid].T
    wm3 = W_M[:, 2 * d_hid :].T
    wu1 = W_U[:, :d_hid].T
    wu2 = W_U[:, d_hid:].T
    wd1 = W_dec[:, :d_hid].T
    wd2 = W_dec[:, d_hid:].T
    wt1 = W_term[:, :d_hid].T
    wt2 = W_term[:, d_hid:].T
    be = b_enc[None, :]
    bm = b_M[None, :]
    bu = b_U[None, :]
    bd = b_dec[None, :]
    bt = b_term[None, :]

    e_rows = _edge_proj(ea_s, wm3, bm)

    state = x
    hidden = jnp.zeros((n, d_hid), jnp.float32)
    preds = [x]
    stops = [jnp.zeros((n, 2), jnp.float32)]
    for _ in range(3):
        z, a_nodes, b_nodes = _tc_encode(state, hidden, wz1, wz2, be, wm1, wm2)
        s_full = _sc_segmax(a_nodes, src_s, dst_s, e_rows, offs, npt)
        s = s_full[:n]
        nh, ns, hsum = _tc_update(s, z, b_nodes, wu1, wu2, bu, wd1, wd2, bd)
        stop = _tc_stop(nh, hsum, wt1, wt2, bt)
        preds.append(ns)
        stops.append(stop)
        state, hidden = ns, nh

    return jnp.stack(preds, axis=1), jnp.stack(stops, axis=1)


# speedup vs baseline: 4.4050x; 1.3213x over previous
import functools

import jax
import jax.numpy as jnp
from jax import lax
from jax.experimental import pallas as pl
from jax.experimental.pallas import tpu as pltpu
from jax.experimental.pallas import tpu_sc as plsc

NC = 2
NS = 16
NW = NC * NS

CHUNK = 128
KEB = 2000
NBLK = 10


def _f32(*shape):
    return jax.ShapeDtypeStruct(shape, jnp.float32)


def _edge_proj(ea_s, wm3t, bm):
    epad = ea_s.shape[0]
    de = ea_s.shape[1]

    def body(ea_ref, w_ref, b_ref, o_ref):
        o_ref[...] = (
            jnp.dot(ea_ref[...], w_ref[...], preferred_element_type=jnp.float32)
            + b_ref[...]
        )

    return pl.pallas_call(
        body,
        grid=(epad // KEB,),
        in_specs=[
            pl.BlockSpec((KEB, de), lambda i: (i, 0)),
            pl.BlockSpec((de, 128), lambda i: (0, 0)),
            pl.BlockSpec((1, 128), lambda i: (0, 0)),
        ],
        out_specs=pl.BlockSpec((KEB, 128), lambda i: (i, 0)),
        out_shape=_f32(epad, 128),
    )(ea_s, wm3t, bm)


def _tc_encode(state, hidden, wz1, wz2, be, wm1, wm2):
    n = state.shape[0]
    blk = n // NBLK

    def body(s_ref, h_ref, wz1_ref, wz2_ref, be_ref, wm1_ref, wm2_ref,
             z_ref, a_ref, b_ref):
        z = (
            jnp.dot(s_ref[...], wz1_ref[...], preferred_element_type=jnp.float32)
            + jnp.dot(h_ref[...], wz2_ref[...], preferred_element_type=jnp.float32)
            + be_ref[...]
        )
        z_ref[...] = z
        a_ref[...] = jnp.dot(z, wm1_ref[...], preferred_element_type=jnp.float32)
        b_ref[...] = jnp.dot(z, wm2_ref[...], preferred_element_type=jnp.float32)

    full = pl.BlockSpec((128, 128), lambda i: (0, 0))
    return pl.pallas_call(
        body,
        grid=(NBLK,),
        in_specs=[
            pl.BlockSpec((blk, 128), lambda i: (i, 0)),
            pl.BlockSpec((blk, 128), lambda i: (i, 0)),
            full, full,
            pl.BlockSpec((1, 128), lambda i: (0, 0)),
            full, full,
        ],
        out_specs=[
            pl.BlockSpec((blk, 128), lambda i: (i, 0)),
            pl.BlockSpec((blk, 128), lambda i: (i, 0)),
            pl.BlockSpec((blk, 128), lambda i: (i, 0)),
        ],
        out_shape=[_f32(n, 128), _f32(n, 128), _f32(n, 128)],
    )(state, hidden, wz1, wz2, be, wm1, wm2)


def _tc_update(s, z, b_rows, wu1, wu2, bu, wd1, wd2, bd):
    n = z.shape[0]
    blk = n // NBLK

    def body(s_ref, z_ref, brow_ref, wu1_ref, wu2_ref, bu_ref,
             wd1_ref, wd2_ref, bd_ref, nh_ref, ns_ref, hsum_ref):
        sblk = s_ref[...]
        u = jnp.where(sblk == -jnp.inf, 0.0, sblk + brow_ref[...])
        nh = (
            jnp.dot(z_ref[...], wu1_ref[...], preferred_element_type=jnp.float32)
            + jnp.dot(u, wu2_ref[...], preferred_element_type=jnp.float32)
            + bu_ref[...]
        )
        nh_ref[...] = nh
        ns_ref[...] = (
            jnp.dot(nh, wd1_ref[...], preferred_element_type=jnp.float32)
            + jnp.dot(z_ref[...], wd2_ref[...], preferred_element_type=jnp.float32)
            + bd_ref[...]
        )

        @pl.when(pl.program_id(0) == 0)
        def _():
            hsum_ref[...] = jnp.zeros_like(hsum_ref)

        hsum_ref[...] += jnp.sum(nh, axis=0, keepdims=True)

    full = pl.BlockSpec((128, 128), lambda i: (0, 0))
    rowspec = pl.BlockSpec((blk, 128), lambda i: (i, 0))
    return pl.pallas_call(
        body,
        grid=(NBLK,),
        in_specs=[
            rowspec, rowspec, rowspec,
            full, full, pl.BlockSpec((1, 128), lambda i: (0, 0)),
            full, full, pl.BlockSpec((1, 128), lambda i: (0, 0)),
        ],
        out_specs=[
            rowspec, rowspec,
            pl.BlockSpec((1, 128), lambda i: (0, 0)),
        ],
        out_shape=[_f32(n, 128), _f32(n, 128), _f32(1, 128)],
    )(s, z, b_rows, wu1, wu2, bu, wd1, wd2, bd)


def _tc_stop(nh, hsum, wt1, wt2, bt):
    n = nh.shape[0]
    blk = n // NBLK
    inv_n = 1.0 / n

    def body(nh_ref, hsum_ref, wt1_ref, wt2_ref, bt_ref, o_ref):
        m = hsum_ref[...] * inv_n
        v = (
            jnp.dot(nh_ref[...], wt1_ref[...], preferred_element_type=jnp.float32)
            + jnp.dot(m, wt2_ref[...], preferred_element_type=jnp.float32)
            + bt_ref[...]
        )
        o_ref[...] = 1.0 / (1.0 + jnp.exp(-v))

    return pl.pallas_call(
        body,
        grid=(NBLK,),
        in_specs=[
            pl.BlockSpec((blk, 128), lambda i: (i, 0)),
            pl.BlockSpec((1, 128), lambda i: (0, 0)),
            pl.BlockSpec((128, 2), lambda i: (0, 0)),
            pl.BlockSpec((128, 2), lambda i: (0, 0)),
            pl.BlockSpec((1, 2), lambda i: (0, 0)),
        ],
        out_specs=pl.BlockSpec((blk, 2), lambda i: (i, 0)),
        out_shape=_f32(n, 2),
    )(nh, hsum, wt1, wt2, bt)


def _sc_segmax(a_nodes, src_s, dst_s, perm_s, e_rows, offs, npt):
    n_pad = NW * npt
    mesh = plsc.VectorSubcoreMesh(core_axis_name="c", subcore_axis_name="s")

    @functools.partial(
        pl.kernel,
        out_type=_f32(n_pad, 128),
        mesh=mesh,
        scratch_types=[
            pltpu.VMEM((48,), jnp.int32),
            [pltpu.VMEM((CHUNK,), jnp.int32)] * 2,
            [pltpu.VMEM((CHUNK + 16,), jnp.int32)] * 2,
            [pltpu.VMEM((CHUNK,), jnp.int32)] * 2,
            [pltpu.VMEM((CHUNK, 128), jnp.float32)] * 2,
            [pltpu.VMEM((CHUNK, 128), jnp.float32)] * 2,
            pltpu.VMEM((npt + 8, 128), jnp.float32),
            [pltpu.SemaphoreType.DMA] * 2,
            [pltpu.SemaphoreType.DMA] * 2,
            [pltpu.SemaphoreType.DMA] * 2,
        ],
    )
    def k(a_hbm, src_hbm, dst_hbm, perm_hbm, e_hbm, off_hbm, out_hbm,
          off_v, src_v, dst_v, prm_v, a_v, e_v, acc, lin_sem, gata_sem,
          gate_sem):
        wid = lax.axis_index("s") * NC + lax.axis_index("c")
        lo = wid * npt
        pltpu.sync_copy(off_hbm, off_v)
        ov = off_v[pl.ds(wid, 16)]
        start = ov[0]
        end = ov[1]
        astart = (start // 8) * 8
        nchunks = (end - astart + CHUNK - 1) // CHUNK

        neg_inf = jnp.full((16,), -jnp.inf, jnp.float32)

        def initrow(i, carry):
            for j in range(8):
                acc[i, pl.ds(j * 16, 16)] = neg_inf
            return carry

        lax.fori_loop(0, npt + 8, initrow, 0)

        init_carry = (jnp.int32(npt), *([neg_inf] * 8))

        def issue_lin(c, b):
            @pl.when(c < nchunks)
            def _():
                base = astart + c * CHUNK
                pltpu.async_copy(src_hbm.at[pl.ds(base, CHUNK)], src_v[b], lin_sem[b])
                pltpu.async_copy(
                    dst_hbm.at[pl.ds(base, CHUNK)], dst_v[b].at[pl.ds(0, CHUNK)],
                    lin_sem[b],
                )
                pltpu.async_copy(perm_hbm.at[pl.ds(base, CHUNK)], prm_v[b], lin_sem[b])

        def wait_lin(c, b):
            @pl.when(c < nchunks)
            def _():
                pltpu.make_async_copy(
                    src_hbm.at[pl.ds(0, CHUNK)], src_v[b], lin_sem[b]
                ).wait()
                pltpu.make_async_copy(
                    dst_hbm.at[pl.ds(0, CHUNK)], dst_v[b].at[pl.ds(0, CHUNK)],
                    lin_sem[b],
                ).wait()
                pltpu.make_async_copy(
                    perm_hbm.at[pl.ds(0, CHUNK)], prm_v[b], lin_sem[b]
                ).wait()

        def issue_gat(c, b):
            @pl.when(c < nchunks)
            def _():
                pltpu.async_copy(a_hbm.at[src_v[b]], a_v[b], gata_sem[b])
                pltpu.async_copy(e_hbm.at[prm_v[b]], e_v[b], gate_sem[b])

        def wait_gat(c, b):
            @pl.when(c < nchunks)
            def _():
                pltpu.make_async_copy(
                    e_hbm.at[pl.ds(0, CHUNK)], a_v[b], gata_sem[b]
                ).wait()
                pltpu.make_async_copy(
                    e_hbm.at[pl.ds(0, CHUNK)], e_v[b], gate_sem[b]
                ).wait()

        def process(c, b, carry):
            base = astart + c * CHUNK
            lb = jnp.maximum(start - base, 0)
            ub = jnp.minimum(end - base, CHUNK)
            dv, av, ev = dst_v[b], a_v[b], e_v[b]

            def edge_body(i, ecarry):
                d_prev = ecarry[0]
                m = ecarry[1:]
                d = dv[pl.ds(i, 16)][0] - lo
                same = d == d_prev

                @pl.when(jnp.logical_not(same))
                def _():
                    for j in range(8):
                        sl = pl.ds(j * 16, 16)
                        acc[d_prev, sl] = jnp.maximum(acc[d_prev, sl], m[j])

                new_m = []
                for j in range(8):
                    sl = pl.ds(j * 16, 16)
                    p = av[i, sl] + ev[i, sl]
                    prev = jnp.where(same, m[j], neg_inf)
                    new_m.append(jnp.maximum(prev, p))
                return (d, *new_m)

            return lax.fori_loop(lb, ub, edge_body, carry)

        issue_lin(0, 0)
        wait_lin(0, 0)
        issue_gat(0, 0)
        issue_lin(1, 1)
        nq = (nchunks + 1) // 2

        def pair_body(q, carry):
            c0 = 2 * q
            wait_gat(c0, 0)
            wait_lin(c0 + 1, 1)
            issue_gat(c0 + 1, 1)
            carry = process(c0, 0, carry)
            issue_lin(c0 + 2, 0)
            wait_gat(c0 + 1, 1)
            wait_lin(c0 + 2, 0)
            issue_gat(c0 + 2, 0)
            carry = process(c0 + 1, 1, carry)
            issue_lin(c0 + 3, 1)
            return carry

        fin = lax.fori_loop(0, nq, pair_body, init_carry)
        d_last = fin[0]
        for j in range(8):
            sl = pl.ds(j * 16, 16)
            acc[d_last, sl] = jnp.maximum(acc[d_last, sl], fin[1 + j])
        pltpu.sync_copy(acc.at[pl.ds(0, npt)], out_hbm.at[pl.ds(lo, npt)])

    return k(a_nodes, src_s, dst_s, perm_s, e_rows, offs)


def kernel(x, edge_index, edge_attr, W_enc, b_enc, W_M, b_M, W_U, b_U,
           W_dec, b_dec, W_term, b_term):
    n = x.shape[0]
    e = edge_attr.shape[0]
    d_hid = W_enc.shape[0]
    npt = ((-(-n // NW) + 7) // 8) * 8

    src = edge_index[0].astype(jnp.int32)
    dst = edge_index[1].astype(jnp.int32)

    iota = jnp.arange(e, dtype=jnp.int32)
    dst_s, src_s, perm = lax.sort((dst, src, iota), num_keys=1)

    e_pad = e + 2 * CHUNK
    big = jnp.int32(1 << 28)
    dst_s = jnp.concatenate([dst_s, jnp.full((e_pad - e,), big, jnp.int32)])
    src_s = jnp.concatenate([src_s, jnp.zeros((e_pad - e,), jnp.int32)])
    perm = jnp.concatenate([perm, jnp.zeros((e_pad - e,), jnp.int32)])
    tile_lo = (jnp.arange(33, dtype=jnp.int32) * npt).astype(jnp.int32)
    offs = jnp.searchsorted(dst_s, tile_lo).astype(jnp.int32)
    offs = jnp.concatenate([offs, jnp.full((15,), e, jnp.int32)])

    wz1 = W_enc[:, :d_hid].T
    wz2 = W_enc[:, d_hid:].T
    wm1 = W_M[:, :d_hid].T
    wm2 = W_M[:, d_hid : 2 * d_hid].T
    wm3 = W_M[:, 2 * d_hid :].T
    wu1 = W_U[:, :d_hid].T
    wu2 = W_U[:, d_hid:].T
    wd1 = W_dec[:, :d_hid].T
    wd2 = W_dec[:, d_hid:].T
    wt1 = W_term[:, :d_hid].T
    wt2 = W_term[:, d_hid:].T
    be = b_enc[None, :]
    bm = b_M[None, :]
    bu = b_U[None, :]
    bd = b_dec[None, :]
    bt = b_term[None, :]

    e_rows = _edge_proj(edge_attr, wm3, bm)

    state = x
    hidden = jnp.zeros((n, d_hid), jnp.float32)
    preds = [x]
    stops = [jnp.zeros((n, 2), jnp.float32)]
    for _ in range(3):
        z, a_nodes, b_nodes = _tc_encode(state, hidden, wz1, wz2, be, wm1, wm2)
        s_full = _sc_segmax(a_nodes, src_s, dst_s, perm, e_rows, offs, npt)
        s = s_full[:n]
        nh, ns, hsum = _tc_update(s, z, b_nodes, wu1, wu2, bu, wd1, wd2, bd)
        stop = _tc_stop(nh, hsum, wt1, wt2, bt)
        preds.append(ns)
        stops.append(stop)
        state, hidden = ns, nh

    return jnp.stack(preds, axis=1), jnp.stack(stops, axis=1)
